# baseline (speedup 1.0000x reference)
import jax
import jax.numpy as jnp
from jax.experimental import pallas as pl
from jax.experimental.pallas import tpu as pltpu

_BLK = 1024
_NEG = -1e30


def _gate_body(x_ref, w1_ref, b1_ref, w2t_ref, b2t_ref, gated_ref, mask_ref):
    x = x_ref[...]
    h = jnp.dot(x, w1_ref[...], preferred_element_type=jnp.float32) + b1_ref[...]
    h = 0.5 * h * (1.0 + jax.lax.erf(h * 0.7071067811865476))
    lt = jax.lax.dot_general(
        w2t_ref[...], h, (((1,), (1,)), ((), ())),
        preferred_element_type=jnp.float32,
    ) + b2t_ref[...]
    m = jnp.max(lt, axis=0, keepdims=True)
    e = jnp.exp(lt - m)
    s = jnp.sum(e, axis=0, keepdims=True)
    g = e / s
    g0 = g[0:1, :]
    g1 = g[1:2, :]
    g2 = g[2:3, :]
    excl2 = (g2 <= g0) & (g2 <= g1)
    excl1 = (~excl2) & (g1 <= g0) & (g1 < g2)
    excl0 = (~excl2) & (~excl1)
    ones = jnp.ones_like(g0)
    zeros = jnp.zeros_like(g0)
    mt = jnp.concatenate(
        [
            jnp.where(excl0, zeros, ones),
            jnp.where(excl1, zeros, ones),
            jnp.where(excl2, zeros, ones),
        ],
        axis=0,
    )
    gt = g[0:3, :] * mt
    gt = gt / (jnp.sum(gt, axis=0, keepdims=True) + 1e-8)
    gated_ref[...] = gt.T
    mask_ref[...] = mt.T


def kernel(combined_pooled_feat, W1, b1, W2, b2):
    n, d_in = combined_pooled_feat.shape
    d_h = W1.shape[1]
    n_out = W2.shape[1]
    w2t = jnp.zeros((8, d_h), jnp.float32).at[0:n_out, :].set(W2.T)
    b2t = jnp.full((8, 1), _NEG, jnp.float32).at[0:n_out, 0].set(b2)
    grid = (n // _BLK,)
    gated, mask = pl.pallas_call(
        _gate_body,
        grid=grid,
        in_specs=[
            pl.BlockSpec((_BLK, d_in), lambda i: (i, 0)),
            pl.BlockSpec((d_in, d_h), lambda i: (0, 0)),
            pl.BlockSpec((1, d_h), lambda i: (0, 0)),
            pl.BlockSpec((8, d_h), lambda i: (0, 0)),
            pl.BlockSpec((8, 1), lambda i: (0, 0)),
        ],
        out_specs=[
            pl.BlockSpec((_BLK, n_out), lambda i: (i, 0)),
            pl.BlockSpec((_BLK, n_out), lambda i: (i, 0)),
        ],
        out_shape=[
            jax.ShapeDtypeStruct((n, n_out), jnp.float32),
            jax.ShapeDtypeStruct((n, n_out), jnp.float32),
        ],
        compiler_params=pltpu.CompilerParams(
            dimension_semantics=("parallel",),
        ),
    )(combined_pooled_feat, W1, b1.reshape(1, d_h), w2t, b2t)
    return (gated, mask)


# speedup vs baseline: 1.1103x; 1.1103x over previous
import jax
import jax.numpy as jnp
from jax.experimental import pallas as pl
from jax.experimental.pallas import tpu as pltpu

_BLK = 2048
_NEG = -1e30


def _gate_body(x_ref, w1_ref, b1_ref, w2t_ref, b2t_ref, gated_ref, mask_ref):
    x = x_ref[...]
    h = jnp.dot(x, w1_ref[...], preferred_element_type=jnp.float32) + b1_ref[...]
    h = 0.5 * h * (1.0 + jax.lax.erf(h * 0.7071067811865476))
    lt = jax.lax.dot_general(
        w2t_ref[...], h, (((1,), (1,)), ((), ())),
        preferred_element_type=jnp.float32,
    ) + b2t_ref[...]
    m = jnp.max(lt, axis=0, keepdims=True)
    e = jnp.exp(lt - m)
    s = jnp.sum(e, axis=0, keepdims=True)
    g = e / s
    g0 = g[0:1, :]
    g1 = g[1:2, :]
    g2 = g[2:3, :]
    excl2 = (g2 <= g0) & (g2 <= g1)
    excl1 = (~excl2) & (g1 <= g0) & (g1 < g2)
    excl0 = (~excl2) & (~excl1)
    ones = jnp.ones_like(g0)
    zeros = jnp.zeros_like(g0)
    mt = jnp.concatenate(
        [
            jnp.where(excl0, zeros, ones),
            jnp.where(excl1, zeros, ones),
            jnp.where(excl2, zeros, ones),
        ],
        axis=0,
    )
    gt = g[0:3, :] * mt
    gt = gt / (jnp.sum(gt, axis=0, keepdims=True) + 1e-8)
    gated_ref[...] = gt.T
    mask_ref[...] = mt.T


def kernel(combined_pooled_feat, W1, b1, W2, b2):
    n, d_in = combined_pooled_feat.shape
    d_h = W1.shape[1]
    n_out = W2.shape[1]
    w2t = jnp.zeros((8, d_h), jnp.float32).at[0:n_out, :].set(W2.T)
    b2t = jnp.full((8, 1), _NEG, jnp.float32).at[0:n_out, 0].set(b2)
    grid = (n // _BLK,)
    gated, mask = pl.pallas_call(
        _gate_body,
        grid=grid,
        in_specs=[
            pl.BlockSpec((_BLK, d_in), lambda i: (i, 0)),
            pl.BlockSpec((d_in, d_h), lambda i: (0, 0)),
            pl.BlockSpec((1, d_h), lambda i: (0, 0)),
            pl.BlockSpec((8, d_h), lambda i: (0, 0)),
            pl.BlockSpec((8, 1), lambda i: (0, 0)),
        ],
        out_specs=[
            pl.BlockSpec((_BLK, n_out), lambda i: (i, 0)),
            pl.BlockSpec((_BLK, n_out), lambda i: (i, 0)),
        ],
        out_shape=[
            jax.ShapeDtypeStruct((n, n_out), jnp.float32),
            jax.ShapeDtypeStruct((n, n_out), jnp.float32),
        ],
        compiler_params=pltpu.CompilerParams(
            dimension_semantics=("parallel",),
        ),
    )(combined_pooled_feat, W1, b1.reshape(1, d_h), w2t, b2t)
    return (gated, mask)


# speedup vs baseline: 1.1702x; 1.0539x over previous
import jax
import jax.numpy as jnp
from jax.experimental import pallas as pl
from jax.experimental.pallas import tpu as pltpu

_BLK = 4096
_NEG = -1e30


def _gate_body(x_ref, w1_ref, b1_ref, w2t_ref, b2t_ref, gated_ref, mask_ref):
    x = x_ref[...]
    h = jnp.dot(x, w1_ref[...], preferred_element_type=jnp.float32) + b1_ref[...]
    h = 0.5 * h * (1.0 + jax.lax.erf(h * 0.7071067811865476))
    lt = jax.lax.dot_general(
        w2t_ref[...], h, (((1,), (1,)), ((), ())),
        preferred_element_type=jnp.float32,
    ) + b2t_ref[...]
    m = jnp.max(lt, axis=0, keepdims=True)
    e = jnp.exp(lt - m)
    s = jnp.sum(e, axis=0, keepdims=True)
    g = e / s
    g0 = g[0:1, :]
    g1 = g[1:2, :]
    g2 = g[2:3, :]
    excl2 = (g2 <= g0) & (g2 <= g1)
    excl1 = (~excl2) & (g1 <= g0) & (g1 < g2)
    excl0 = (~excl2) & (~excl1)
    ones = jnp.ones_like(g0)
    zeros = jnp.zeros_like(g0)
    mt = jnp.concatenate(
        [
            jnp.where(excl0, zeros, ones),
            jnp.where(excl1, zeros, ones),
            jnp.where(excl2, zeros, ones),
        ],
        axis=0,
    )
    gt = g[0:3, :] * mt
    gt = gt / (jnp.sum(gt, axis=0, keepdims=True) + 1e-8)
    gated_ref[...] = gt.T
    mask_ref[...] = mt.T


def kernel(combined_pooled_feat, W1, b1, W2, b2):
    n, d_in = combined_pooled_feat.shape
    d_h = W1.shape[1]
    n_out = W2.shape[1]
    w2t = jnp.zeros((8, d_h), jnp.float32).at[0:n_out, :].set(W2.T)
    b2t = jnp.full((8, 1), _NEG, jnp.float32).at[0:n_out, 0].set(b2)
    grid = (n // _BLK,)
    gated, mask = pl.pallas_call(
        _gate_body,
        grid=grid,
        in_specs=[
            pl.BlockSpec((_BLK, d_in), lambda i: (i, 0)),
            pl.BlockSpec((d_in, d_h), lambda i: (0, 0)),
            pl.BlockSpec((1, d_h), lambda i: (0, 0)),
            pl.BlockSpec((8, d_h), lambda i: (0, 0)),
            pl.BlockSpec((8, 1), lambda i: (0, 0)),
        ],
        out_specs=[
            pl.BlockSpec((_BLK, n_out), lambda i: (i, 0)),
            pl.BlockSpec((_BLK, n_out), lambda i: (i, 0)),
        ],
        out_shape=[
            jax.ShapeDtypeStruct((n, n_out), jnp.float32),
            jax.ShapeDtypeStruct((n, n_out), jnp.float32),
        ],
        compiler_params=pltpu.CompilerParams(
            dimension_semantics=("parallel",),
        ),
    )(combined_pooled_feat, W1, b1.reshape(1, d_h), w2t, b2t)
    return (gated, mask)


# speedup vs baseline: 1.2134x; 1.0369x over previous
import jax
import jax.numpy as jnp
from jax.experimental import pallas as pl
from jax.experimental.pallas import tpu as pltpu

_BLK = 8192
_NEG = -1e30


def _gate_body(x_ref, w1_ref, b1_ref, w2t_ref, b2t_ref, gated_ref, mask_ref):
    x = x_ref[...]
    h = jnp.dot(x, w1_ref[...], preferred_element_type=jnp.float32) + b1_ref[...]
    h = 0.5 * h * (1.0 + jax.lax.erf(h * 0.7071067811865476))
    lt = jax.lax.dot_general(
        w2t_ref[...], h, (((1,), (1,)), ((), ())),
        preferred_element_type=jnp.float32,
    ) + b2t_ref[...]
    m = jnp.max(lt, axis=0, keepdims=True)
    e = jnp.exp(lt - m)
    s = jnp.sum(e, axis=0, keepdims=True)
    g = e / s
    g0 = g[0:1, :]
    g1 = g[1:2, :]
    g2 = g[2:3, :]
    excl2 = (g2 <= g0) & (g2 <= g1)
    excl1 = (~excl2) & (g1 <= g0) & (g1 < g2)
    excl0 = (~excl2) & (~excl1)
    ones = jnp.ones_like(g0)
    zeros = jnp.zeros_like(g0)
    mt = jnp.concatenate(
        [
            jnp.where(excl0, zeros, ones),
            jnp.where(excl1, zeros, ones),
            jnp.where(excl2, zeros, ones),
        ],
        axis=0,
    )
    gt = g[0:3, :] * mt
    gt = gt / (jnp.sum(gt, axis=0, keepdims=True) + 1e-8)
    gated_ref[...] = gt.T
    mask_ref[...] = mt.T


def kernel(combined_pooled_feat, W1, b1, W2, b2):
    n, d_in = combined_pooled_feat.shape
    d_h = W1.shape[1]
    n_out = W2.shape[1]
    w2t = jnp.zeros((8, d_h), jnp.float32).at[0:n_out, :].set(W2.T)
    b2t = jnp.full((8, 1), _NEG, jnp.float32).at[0:n_out, 0].set(b2)
    grid = (n // _BLK,)
    gated, mask = pl.pallas_call(
        _gate_body,
        grid=grid,
        in_specs=[
            pl.BlockSpec((_BLK, d_in), lambda i: (i, 0)),
            pl.BlockSpec((d_in, d_h), lambda i: (0, 0)),
            pl.BlockSpec((1, d_h), lambda i: (0, 0)),
            pl.BlockSpec((8, d_h), lambda i: (0, 0)),
            pl.BlockSpec((8, 1), lambda i: (0, 0)),
        ],
        out_specs=[
            pl.BlockSpec((_BLK, n_out), lambda i: (i, 0)),
            pl.BlockSpec((_BLK, n_out), lambda i: (i, 0)),
        ],
        out_shape=[
            jax.ShapeDtypeStruct((n, n_out), jnp.float32),
            jax.ShapeDtypeStruct((n, n_out), jnp.float32),
        ],
        compiler_params=pltpu.CompilerParams(
            dimension_semantics=("parallel",),
        ),
    )(combined_pooled_feat, W1, b1.reshape(1, d_h), w2t, b2t)
    return (gated, mask)
